# baseline (speedup 1.0000x reference)
import functools

import jax
import jax.numpy as jnp
from jax import lax
from jax.experimental import pallas as pl
from jax.experimental.pallas import tpu as pltpu
from jax.experimental.pallas import tpu_sc as plsc

N_TOKENS = 9216
DIM = 256
N_CODES = 8192

ROW_BLOCK = 256
N_ROW_BLOCKS = N_TOKENS // ROW_BLOCK

NC, NS = 2, 16
NW = NC * NS
B_PER_W = N_TOKENS // NW


def _argmin_body(x_ref, z2_ref, e2_ref, emb_ref, ind_ref):
    scores = lax.dot_general(
        x_ref[...], emb_ref[...], (((1,), (0,)), ((), ())),
        preferred_element_type=jnp.float32)
    dist = z2_ref[...] - 2.0 * scores + e2_ref[...]
    neg = -dist
    m = jnp.max(neg, axis=1, keepdims=True)
    ids = lax.broadcasted_iota(jnp.int32, neg.shape, 1)
    ind_ref[...] = jnp.min(
        jnp.where(neg == m, ids, jnp.int32(N_CODES)), axis=1, keepdims=True)


_argmin_call = pl.pallas_call(
    _argmin_body,
    grid=(N_ROW_BLOCKS,),
    in_specs=[
        pl.BlockSpec((ROW_BLOCK, DIM), lambda i: (i, 0)),
        pl.BlockSpec((ROW_BLOCK, 1), lambda i: (i, 0)),
        pl.BlockSpec((1, N_CODES), lambda i: (0, 0)),
        pl.BlockSpec((DIM, N_CODES), lambda i: (0, 0)),
    ],
    out_specs=pl.BlockSpec((ROW_BLOCK, 1), lambda i: (i, 0)),
    out_shape=jax.ShapeDtypeStruct((N_TOKENS, 1), jnp.int32),
)


N_CHUNK = 3
CHUNK = B_PER_W // N_CHUNK


@functools.cache
def _make_sc_gather():
    mesh = plsc.VectorSubcoreMesh(
        core_axis_name="c", subcore_axis_name="s", num_cores=NC)

    @functools.partial(
        pl.kernel,
        mesh=mesh,
        out_type=jax.ShapeDtypeStruct((N_TOKENS, DIM), jnp.float32),
        scratch_types=[
            pltpu.VMEM((N_CHUNK, CHUNK), jnp.int32),
            pltpu.VMEM((B_PER_W, DIM), jnp.float32),
            pltpu.SemaphoreType.DMA,
        ],
    )
    def _sc_gather(table_hbm, idx_hbm, out_hbm, idx_v, rows_v, sem):
        wid = lax.axis_index("s") * NC + lax.axis_index("c")
        pltpu.sync_copy(idx_hbm.at[wid], idx_v)
        copies = [
            pltpu.async_copy(table_hbm.at[idx_v.at[j]],
                             rows_v.at[pl.ds(j * CHUNK, CHUNK)], sem)
            for j in range(N_CHUNK)
        ]
        for c in copies:
            c.wait()
        pltpu.sync_copy(rows_v, out_hbm.at[pl.ds(wid * B_PER_W, B_PER_W)])

    return _sc_gather


def _finish_body(x_ref, q_ref, out_ref, acc_ref):
    i = pl.program_id(0)
    x = x_ref[...]
    d = q_ref[...] - x
    out_ref[...] = x + d
    part = jnp.sum(d * d)

    @pl.when(i == 0)
    def _():
        acc_ref[0, 0] = part

    @pl.when(i > 0)
    def _():
        acc_ref[0, 0] += part

    @pl.when(i == pl.num_programs(0) - 1)
    def _():
        acc_ref[0, 0] = acc_ref[0, 0] / jnp.float32(N_TOKENS * DIM)


_finish_call = pl.pallas_call(
    _finish_body,
    grid=(N_ROW_BLOCKS,),
    in_specs=[
        pl.BlockSpec((ROW_BLOCK, DIM), lambda i: (i, 0)),
        pl.BlockSpec((ROW_BLOCK, DIM), lambda i: (i, 0)),
    ],
    out_specs=[
        pl.BlockSpec((ROW_BLOCK, DIM), lambda i: (i, 0)),
        pl.BlockSpec((1, 1), lambda i: (0, 0), memory_space=pltpu.SMEM),
    ],
    out_shape=[
        jax.ShapeDtypeStruct((N_TOKENS, DIM), jnp.float32),
        jax.ShapeDtypeStruct((1, 1), jnp.float32),
    ],
)


def kernel(input, embed):
    b, s, d = input.shape
    flatten = input.reshape(-1, d)
    z2 = jnp.sum(flatten ** 2, axis=1, keepdims=True)
    e2 = jnp.sum(embed ** 2, axis=0, keepdims=True)
    ind2d = _argmin_call(flatten, z2, e2, embed)
    idx = ind2d.reshape(NW, N_CHUNK, CHUNK)
    quantize = _make_sc_gather()(jnp.transpose(embed), idx)
    qst, diff = _finish_call(flatten, quantize)
    return (qst.reshape(b, s, d), diff[0, 0], ind2d.reshape(b, s, 1))
